# Initial kernel scaffold; baseline (speedup 1.0000x reference)
import jax
import jax.numpy as jnp
from jax.experimental import pallas as pl


def kernel(logits):
    raise NotImplementedError("write your pallas kernel here")


# speedup vs baseline: 6.9789x; 6.9789x over previous
import functools

import jax
import jax.numpy as jnp
from jax import lax
from jax.experimental import pallas as pl
from jax.experimental.pallas import tpu as pltpu
from jax.experimental.pallas import tpu_sc as plsc

N_ROWS = 128
N_VOCAB = 100000
L = 16
NV = N_VOCAB // L
B1 = 4096
B2 = 4096
S1 = 256.0
S2 = S1 * B2
TOP_P = 0.95
NC = 2
NS = 16
ROWS_PER = N_ROWS // (NC * NS)


def _nucleus_body(x_hbm, out_hbm, row_v, h1_v, h2_v):
    wid = lax.axis_index("s") * NC + lax.axis_index("c")

    def per_row(r, _carry):
        row = wid * ROWS_PER + r
        pltpu.sync_copy(x_hbm.at[row], row_v)

        def p1(i, acc):
            return jnp.maximum(acc, row_v[pl.ds(i * L, L)])

        mvec = lax.fori_loop(0, NV, p1, jnp.full((L,), -jnp.inf, jnp.float32))
        m = jnp.max(mvec)

        def z1(i, c):
            h1_v[pl.ds(i * L, L)] = jnp.zeros((L,), jnp.float32)
            return c

        lax.fori_loop(0, B1 // L, z1, 0)

        def p2(i, zacc):
            x = row_v[pl.ds(i * L, L)]
            u = m - x
            e = jnp.exp(-u)
            bf = jnp.minimum(u * S1, float(B1 - 1))
            b = bf.astype(jnp.int32)
            plsc.addupdate_scatter(h1_v, [b], e)
            return zacc + e

        zvec = lax.fori_loop(0, NV, p2, jnp.zeros((L,), jnp.float32))
        target = TOP_P * jnp.sum(zvec)

        def p3(i, carry):
            run, cnt, mb = carry
            h = h1_v[pl.ds(i * L, L)]
            c = plsc.cumsum(h) + run
            lt = c < target
            cnt = cnt + jnp.sum(jnp.where(lt, 1, 0).astype(jnp.int32))
            mb = mb + jnp.sum(jnp.where(lt, h, 0.0))
            return run + jnp.sum(h), cnt, mb

        _, cnt1, mass_before = lax.fori_loop(
            0, B1 // L, p3,
            (jnp.float32(0.0), jnp.int32(0), jnp.float32(0.0)))
        bstar = jnp.minimum(cnt1, B1 - 1)
        u1 = bstar.astype(jnp.float32) * (1.0 / S1)

        def z2(i, c):
            h2_v[pl.ds(i * L, L)] = jnp.zeros((L,), jnp.float32)
            return c

        lax.fori_loop(0, B2 // L, z2, 0)

        def p4(i, c):
            x = row_v[pl.ds(i * L, L)]
            u = m - x
            e = jnp.exp(-u)
            bf = jnp.minimum(u * S1, float(B1 - 1))
            b = bf.astype(jnp.int32)
            msk = b == bstar
            b2f = jnp.clip((u - u1) * S2, 0.0, float(B2 - 1))
            b2 = b2f.astype(jnp.int32)
            plsc.addupdate_scatter(h2_v, [b2], e, mask=msk)
            return c

        lax.fori_loop(0, NV, p4, 0)
        target2 = target - mass_before

        def p5(i, carry):
            run, cnt, sin = carry
            h = h2_v[pl.ds(i * L, L)]
            c = plsc.cumsum(h) + run
            cnt = cnt + jnp.sum(jnp.where(c < target2, 1, 0).astype(jnp.int32))
            sin = sin + jnp.sum(jnp.where((c - h) < target2, h, 0.0))
            return run + jnp.sum(h), cnt, sin

        _, b2star, s_in = lax.fori_loop(
            0, B2 // L, p5,
            (jnp.float32(0.0), jnp.int32(0), jnp.float32(0.0)))
        inv_s = jnp.ones((L,), jnp.float32) / (
            jnp.zeros((L,), jnp.float32) + (mass_before + s_in))

        def p6(i, c):
            sl = pl.ds(i * L, L)
            x = row_v[sl]
            u = m - x
            e = jnp.exp(-u)
            bf = jnp.minimum(u * S1, float(B1 - 1))
            b = bf.astype(jnp.int32)
            b2 = jnp.clip((u - u1) * S2, 0.0, float(B2 - 1)).astype(jnp.int32)
            keep = (b < bstar) | ((b == bstar) & (b2 <= b2star))
            row_v[sl] = jnp.where(keep, e * inv_s, 0.0)
            return c

        lax.fori_loop(0, NV, p6, 0)
        pltpu.sync_copy(row_v, out_hbm.at[row])
        return _carry

    lax.fori_loop(0, ROWS_PER, per_row, 0)


@jax.jit
def _nucleus_sc(logits):
    mesh = plsc.VectorSubcoreMesh(core_axis_name="c", subcore_axis_name="s")
    f = functools.partial(
        pl.kernel,
        mesh=mesh,
        out_type=jax.ShapeDtypeStruct((N_ROWS, N_VOCAB), jnp.float32),
        scratch_types=[
            pltpu.VMEM((N_VOCAB,), jnp.float32),
            pltpu.VMEM((B1,), jnp.float32),
            pltpu.VMEM((B2,), jnp.float32),
        ],
        compiler_params=pltpu.CompilerParams(needs_layout_passes=False),
    )(_nucleus_body)
    return f(logits)


def kernel(logits):
    return _nucleus_sc(logits)


# speedup vs baseline: 10.2888x; 1.4743x over previous
import functools

import jax
import jax.numpy as jnp
from jax import lax
from jax.experimental import pallas as pl
from jax.experimental.pallas import tpu as pltpu
from jax.experimental.pallas import tpu_sc as plsc

N_ROWS = 128
N_VOCAB = 100000
L = 16
NV = N_VOCAB // L
B1 = 4096
B2 = 1024
S1 = 256.0
S2 = S1 * B2
TOP_P = 0.95
NC = 2
NS = 16
ROWS_PER = N_ROWS // (NC * NS)


def _bf_sum(v):
    idx = lax.iota(jnp.int32, L)
    for s in (8, 4, 2, 1):
        v = v + jnp.take(v, jnp.bitwise_xor(idx, s))
    return v


def _bf_max(v):
    idx = lax.iota(jnp.int32, L)
    for s in (8, 4, 2, 1):
        v = jnp.maximum(v, jnp.take(v, jnp.bitwise_xor(idx, s)))
    return v


def _nucleus_body(x_hbm, out_hbm, row_v, h1_v, h2_v):
    wid = lax.axis_index("s") * NC + lax.axis_index("c")
    zeros = jnp.zeros((L,), jnp.float32)
    ones = jnp.ones((L,), jnp.float32)
    izeros = jnp.zeros((L,), jnp.int32)
    iones = jnp.ones((L,), jnp.int32)

    def per_row(r, _carry):
        row = wid * ROWS_PER + r
        pltpu.sync_copy(x_hbm.at[row], row_v)

        U1 = 10
        def p1(i, accs):
            a = list(accs)
            for j in range(U1):
                a[j % 4] = jnp.maximum(a[j % 4], row_v[pl.ds((i * U1 + j) * L, L)])
            return tuple(a)

        neg = jnp.full((L,), -jnp.inf, jnp.float32)
        a0, a1, a2, a3 = lax.fori_loop(0, NV // U1, p1, (neg, neg, neg, neg))
        m = _bf_max(jnp.maximum(jnp.maximum(a0, a1), jnp.maximum(a2, a3)))

        def z1(i, c):
            for j in range(16):
                h1_v[pl.ds((i * 16 + j) * L, L)] = zeros
            return c

        lax.fori_loop(0, B1 // (16 * L), z1, 0)

        U2 = 5
        def p2(i, c):
            for j in range(U2):
                x = row_v[pl.ds((i * U2 + j) * L, L)]
                u = m - x
                e = jnp.exp(x - m)
                b = jnp.minimum(u * S1, float(B1 - 1)).astype(jnp.int32)
                plsc.addupdate_scatter(h1_v, [b], e)
            return c

        lax.fori_loop(0, NV // U2, p2, 0)

        def p3a(i, accs):
            a = list(accs)
            for j in range(8):
                a[j % 4] = a[j % 4] + h1_v[pl.ds((i * 8 + j) * L, L)]
            return tuple(a)

        s0, s1_, s2_, s3 = lax.fori_loop(
            0, B1 // (8 * L), p3a, (zeros, zeros, zeros, zeros))
        target = TOP_P * _bf_sum((s0 + s1_) + (s2_ + s3))

        def p3b(i, carry):
            run, cnt, mb = carry
            h = h1_v[pl.ds(i * L, L)]
            c = plsc.cumsum(h) + run
            lt = c < target
            cnt = cnt + jnp.where(lt, iones, izeros)
            mb = mb + jnp.where(lt, h, zeros)
            return run + _bf_sum(h), cnt, mb

        _, cntv, mbv = lax.fori_loop(0, B1 // L, p3b, (zeros, izeros, zeros))
        bstar = jnp.minimum(_bf_sum(cntv), B1 - 1)
        mass_before = _bf_sum(mbv)
        u1 = bstar.astype(jnp.float32) * (1.0 / S1)

        def z2(i, c):
            for j in range(16):
                h2_v[pl.ds((i * 16 + j) * L, L)] = zeros
            return c

        lax.fori_loop(0, B2 // (16 * L), z2, 0)

        U4 = 5
        def p4(i, c):
            for j in range(U4):
                x = row_v[pl.ds((i * U4 + j) * L, L)]
                u = m - x
                e = jnp.exp(x - m)
                b = jnp.minimum(u * S1, float(B1 - 1)).astype(jnp.int32)
                msk = b == bstar
                b2f = jnp.clip((u - u1) * S2, 0.0, float(B2 - 1))
                plsc.addupdate_scatter(h2_v, [b2f.astype(jnp.int32)], e, mask=msk)
            return c

        lax.fori_loop(0, NV // U4, p4, 0)
        target2 = target - mass_before

        def p5(i, carry):
            run, cnt, sin = carry
            h = h2_v[pl.ds(i * L, L)]
            c = plsc.cumsum(h) + run
            cnt = cnt + jnp.where(c < target2, iones, izeros)
            sin = sin + jnp.where((c - h) < target2, h, zeros)
            return run + _bf_sum(h), cnt, sin

        _, cnt2v, sinv = lax.fori_loop(0, B2 // L, p5, (zeros, izeros, zeros))
        b2star = _bf_sum(cnt2v)
        u_cut = u1 + (b2star + 1).astype(jnp.float32) * (1.0 / S2)
        inv_s = ones / (zeros + (mass_before + _bf_sum(sinv)))

        U6 = 10
        def p6(i, c):
            for j in range(U6):
                sl = pl.ds((i * U6 + j) * L, L)
                x = row_v[sl]
                u = m - x
                e = jnp.exp(x - m)
                row_v[sl] = jnp.where(u < u_cut, e * inv_s, zeros)
            return c

        lax.fori_loop(0, NV // U6, p6, 0)
        pltpu.sync_copy(row_v, out_hbm.at[row])
        return _carry

    lax.fori_loop(0, ROWS_PER, per_row, 0)


@jax.jit
def _nucleus_sc(logits):
    mesh = plsc.VectorSubcoreMesh(core_axis_name="c", subcore_axis_name="s")
    f = functools.partial(
        pl.kernel,
        mesh=mesh,
        out_type=jax.ShapeDtypeStruct((N_ROWS, N_VOCAB), jnp.float32),
        scratch_types=[
            pltpu.VMEM((N_VOCAB,), jnp.float32),
            pltpu.VMEM((B1,), jnp.float32),
            pltpu.VMEM((B2,), jnp.float32),
        ],
        compiler_params=pltpu.CompilerParams(needs_layout_passes=False),
    )(_nucleus_body)
    return f(logits)


def kernel(logits):
    return _nucleus_sc(logits)
